# Initial kernel scaffold; baseline (speedup 1.0000x reference)
import jax
import jax.numpy as jnp
from jax.experimental import pallas as pl


def kernel(boxes, scores, labels, masks):
    raise NotImplementedError("write your pallas kernel here")


# speedup vs baseline: 5.5055x; 5.5055x over previous
import jax
import jax.numpy as jnp
from jax.experimental import pallas as pl
from jax.experimental.pallas import tpu as pltpu

SCORE_T = 0.5
NMS_T = 0.5


def _nms_body(b_ref, bT_ref, sc_ref, sr_ref, lab_ref,
              bx_ref, lb_ref, so_ref, kp_ref, pm_ref, src_ref,
              iou_ref):
    n = b_ref.shape[0]
    i0 = jax.lax.broadcasted_iota(jnp.int32, (n, n), 0)
    i1 = jax.lax.broadcasted_iota(jnp.int32, (n, n), 1)
    sa = sc_ref[...]
    sb = sr_ref[...]

    cmp_ij = (sb > sa) | ((sb == sa) & (i1 < i0))
    rank_c = jnp.sum(cmp_ij.astype(jnp.int32), axis=1, keepdims=True)
    cmp_ji = (sa > sb) | ((sa == sb) & (i0 < i1))
    rank_r = jnp.sum(cmp_ji.astype(jnp.int32), axis=0, keepdims=True)

    eq = rank_r == i0
    eqT = rank_c == i1
    eqf = eq.astype(jnp.float32)
    eqTf = eqT.astype(jnp.float32)

    def gcol(row_vals):
        return jnp.sum(eqf * row_vals, axis=1, keepdims=True)

    def grow(col_vals):
        return jnp.sum(eqTf * col_vals, axis=0, keepdims=True)

    x1_c = gcol(bT_ref[0:1, :]); y1_c = gcol(bT_ref[1:2, :])
    x2_c = gcol(bT_ref[2:3, :]); y2_c = gcol(bT_ref[3:4, :])
    x1_r = grow(b_ref[:, 0:1]); y1_r = grow(b_ref[:, 1:2])
    x2_r = grow(b_ref[:, 2:3]); y2_r = grow(b_ref[:, 3:4])
    s_c = gcol(sb)
    s_r = grow(sa)
    lab_c = jnp.sum(eq.astype(jnp.int32) * lab_ref[...], axis=1, keepdims=True)
    perm_c = jnp.sum(eq.astype(jnp.int32) * i1, axis=1, keepdims=True)
    perm_r = jnp.sum(eqT.astype(jnp.int32) * i0, axis=0, keepdims=True)

    xx1 = jnp.maximum(x1_c, x1_r)
    yy1 = jnp.maximum(y1_c, y1_r)
    xx2 = jnp.minimum(x2_c, x2_r)
    yy2 = jnp.minimum(y2_c, y2_r)
    inter = jnp.maximum(xx2 - xx1, 0.0) * jnp.maximum(yy2 - yy1, 0.0)
    area_c = (x2_c - x1_c) * (y2_c - y1_c)
    area_r = (x2_r - x1_r) * (y2_r - y1_r)
    iou_ref[...] = inter / (area_c + area_r - inter)

    valid = s_r > SCORE_T
    kcount = jnp.sum(valid.astype(jnp.int32))
    nchunks = (kcount + 7) // 8
    idxr = jax.lax.broadcasted_iota(jnp.int32, (1, n), 1)

    def chunk_body(c, keep):
        base = pl.multiple_of(c * 8, 8)
        chunk = iou_ref[pl.ds(base, 8), :]
        for t in range(8):
            i = c * 8 + t
            row = chunk[t:t + 1, :]
            sup = jnp.any((idxr < i) & (keep != 0) & (row > NMS_T))
            keep = jnp.where((idxr == i) & sup, 0, keep)
        return keep

    keep_i = jax.lax.fori_loop(0, nchunks, chunk_body,
                               valid.astype(jnp.int32))
    keep = keep_i != 0

    keep_ci = jnp.sum(((i0 == i1) & keep).astype(jnp.int32),
                      axis=1, keepdims=True)
    keep_cf = keep_ci.astype(jnp.float32)

    bx_ref[:, 0:1] = x1_c * keep_cf
    bx_ref[:, 1:2] = y1_c * keep_cf
    bx_ref[:, 2:3] = x2_c * keep_cf
    bx_ref[:, 3:4] = y2_c * keep_cf
    lb_ref[...] = lab_c * keep_ci
    so_ref[...] = s_c * keep_cf
    kp_ref[...] = keep_ci
    pm_ref[...] = perm_c

    t_col = jnp.max(jnp.where((i1 <= i0) & keep, i1, -1),
                    axis=1, keepdims=True)
    sel = (i1 == t_col).astype(jnp.int32)
    src_col = jnp.sum(sel * perm_r, axis=1, keepdims=True)
    src_ref[...] = jnp.where(t_col < 0, perm_c, src_col)


def _mask_body(src_ref, keep_ref, m_ref, o_ref):
    r = pl.program_id(0)
    o_ref[...] = m_ref[...] * keep_ref[r].astype(jnp.float32)


def kernel(boxes, scores, labels, masks):
    n = boxes.shape[0]
    h, w = masks.shape[2], masks.shape[3]

    bx, lb, so, kp, pm, src = pl.pallas_call(
        _nms_body,
        out_shape=[
            jax.ShapeDtypeStruct((n, 4), jnp.float32),
            jax.ShapeDtypeStruct((n, 1), jnp.int32),
            jax.ShapeDtypeStruct((n, 1), jnp.float32),
            jax.ShapeDtypeStruct((n, 1), jnp.int32),
            jax.ShapeDtypeStruct((n, 1), jnp.int32),
            jax.ShapeDtypeStruct((n, 1), jnp.int32),
        ],
        scratch_shapes=[pltpu.VMEM((n, n), jnp.float32)],
        name="nms_sort",
    )(boxes, boxes.T, scores[:, None], scores[None, :], labels[None, :])

    keep_i = kp[:, 0]
    masks_out = pl.pallas_call(
        _mask_body,
        grid_spec=pltpu.PrefetchScalarGridSpec(
            num_scalar_prefetch=2,
            grid=(n,),
            in_specs=[pl.BlockSpec((1, h, w),
                                   lambda r, src, keep: (src[r], 0, 0))],
            out_specs=pl.BlockSpec((1, h, w),
                                   lambda r, src, keep: (r, 0, 0)),
        ),
        out_shape=jax.ShapeDtypeStruct((n, h, w), jnp.float32),
        compiler_params=pltpu.CompilerParams(
            dimension_semantics=("arbitrary",)),
        name="mask_gather",
    )(src[:, 0], keep_i, masks.reshape(n, h, w))

    return (bx, lb[:, 0], so[:, 0], masks_out.reshape(masks.shape),
            keep_i.astype(jnp.bool_))
